# baseline (speedup 1.0000x reference)
import functools

import jax
import jax.numpy as jnp
from jax import lax
from jax.experimental import pallas as pl
from jax.experimental.pallas import tpu as pltpu
from jax.experimental.pallas import tpu_sc as plsc

N = 10000
E = 320000
H = 128
NUM_OD = 50000
F32 = jnp.float32

NC, NS = 2, 16
NW = NC * NS
N_PAD = 10240
ROWS_PER_TILE = N_PAD // NS

E_ECH = 80
E_NCH = E // E_ECH
E_PER_TILE = E_NCH // NW

CNT_ECH = 128
CNT_NCH = E // CNT_ECH
CNT_ITERS = -(-CNT_NCH // NW)

OD_ECH = 128
OD_PAD = 50048
OD_CHUNKS = OD_PAD // OD_ECH
OD_ITERS = -(-OD_CHUNKS // NW)

NB = 400
EB = 512
CB = 400

def _mesh():
    return plsc.VectorSubcoreMesh(
        core_axis_name="c", subcore_axis_name="s",
        num_cores=NC, num_subcores=NS)


def _dot(a, b):
    return jnp.dot(a, b, preferred_element_type=F32)


def _a_node_body(nf, wnp, bnp, w1a, b1, h_out, hw_out):
    h = jnp.maximum(_dot(nf[...], wnp[...]) + bnp[...], 0.0)
    h_out[...] = h
    hw_out[...] = _dot(h, w1a[...]) + b1[...]


def _a_edge_body(ef, wep, bep, wb, out):
    e = jnp.maximum(_dot(ef[...], wep[...]) + bep[...], 0.0)
    out[...] = _dot(e, wb[...])


def _layer_update(h, acc, cnt, w2, b2, wua, wub, bu, g, beta):
    accsum = acc[0] + acc[1]
    c = cnt[0, :, 0:1] + cnt[1, :, 0:1]
    agg = (_dot(accsum, w2[...]) + c * b2[...]) / (c + 1e-8)
    upd = jnp.maximum(_dot(h, wua[...]) + _dot(agg, wub[...]) + bu[...], 0.0)
    x = upd + h
    mu = jnp.mean(x, axis=-1, keepdims=True)
    xc = x - mu
    var = jnp.mean(xc * xc, axis=-1, keepdims=True)
    return xc * lax.rsqrt(var + 1e-5) * g[...] + beta[...]


def _b_mid_body(h, acc, cnt, w2, b2, wua, wub, bu, g, beta, wn, bn,
                h_out, p_out):
    hn = _layer_update(h[...], acc[...], cnt[...], w2, b2, wua, wub, bu, g, beta)
    h_out[...] = hn
    p_out[...] = _dot(hn, wn[...]) + bn[...]


def _b_last_body(h, acc, cnt, w2, b2, wua, wub, bu, g, beta, ws, wd,
                 h_out, ps_out, pd_out):
    hn = _layer_update(h[...], acc[...], cnt[...], w2, b2, wua, wub, bu, g, beta)
    h_out[...] = hn
    ps_out[...] = _dot(hn, ws[...])
    pd_out[...] = _dot(hn, wd[...])


def _c_od_body(odsum, odf, w1o, b1, w2, b2, w3, b3, out):
    g1 = jnp.maximum(odsum[...] + _dot(odf[...], w1o[...]) + b1[...], 0.0)
    g2 = jnp.maximum(_dot(g1, w2[...]) + b2[...], 0.0)
    corr = jnp.sum(g2 * w3[...][0:1, :], axis=1, keepdims=True)
    out[...] = corr * jnp.ones((1, 8), F32) + b3[...]


def _d_final_body(h, corr2, bn2, fl2, bw1, bb1, bw2, bb2,
                  cw1, cb1, cw2, cb2, la, out):
    ge = jnp.sum(h[...], axis=0, keepdims=True) * (1.0 / N)
    t = jnp.maximum(_dot(ge, bw1[...]) + bb1[...], 0.0)
    logits = _dot(t, bw2[...]) + bb2[...]
    l0 = logits[0, 0]
    l1 = logits[0, 1]
    m = jnp.maximum(l0, l1)
    e0 = jnp.exp(l0 - m)
    e1 = jnp.exp(l1 - m)
    w0 = e0 / (e0 + e1)
    w1 = e1 / (e0 + e1)
    t2 = jnp.maximum(_dot(ge, cw1[...]) + cb1[...], 0.0)
    cv = _dot(t2, cw2[...]) + cb2[...]
    conf = 1.0 / (1.0 + jnp.exp(-cv[0, 0]))
    alpha = jnp.clip(jnp.exp(la[0, 0]), 0.0, 5.0)
    bnv = bn2[...]
    flv = fl2[...]
    crv = corr2[...]
    mb = jnp.max(jnp.abs(bnv))
    mf = jnp.max(jnp.abs(flv))
    mc = jnp.max(jnp.abs(crv))
    out[...] = (w0 * (bnv / (mb + 1e-12)) + w1 * (flv / (mf + 1e-12))
                + (conf * alpha) * (crv / (mc + 1e-12)))


def _zero_fill_rows(ref, nrows, ncols):
    z = jnp.zeros((16,), F32)

    def body(r, _):
        for jj in range(ncols // 16):
            ref[r, pl.ds(jj * 16, 16)] = z
        return 0

    lax.fori_loop(0, nrows, body, 0)


def _sc_edge_body(hw, ew, src, dst, acc_out,
                  g0, e0, g1, e1, s0, d0, s1, d1, s2, d2, s3, d3,
                  acc_sh, sg0, se0, sg1, se1, si0, si1, si2, si3):
    c = lax.axis_index("c")
    s = lax.axis_index("s")
    w = c * NS + s
    row0 = s * ROWS_PER_TILE
    G = (g0, g1)
    EV = (e0, e1)
    SV = (s0, s1, s2, s3)
    DV = (d0, d1, d2, d3)
    SG = (sg0, sg1)
    SE = (se0, se1)
    SI = (si0, si1, si2, si3)

    _zero_fill_rows(g0, E_ECH, H)
    for j in range(ROWS_PER_TILE // E_ECH):
        pltpu.sync_copy(g0, acc_sh.at[pl.ds(row0 + j * E_ECH, E_ECH)])
    plsc.subcore_barrier()

    def prefetch(i, slot):
        off = (w + NW * i) * E_ECH
        pltpu.async_copy(src.at[pl.ds(off, E_ECH)], SV[slot], SI[slot])
        pltpu.async_copy(dst.at[pl.ds(off, E_ECH)], DV[slot], SI[slot])

    def issue(i, slot, b):
        pltpu.make_async_copy(src.at[pl.ds(0, E_ECH)], SV[slot], SI[slot]).wait()
        pltpu.make_async_copy(src.at[pl.ds(0, E_ECH)], DV[slot], SI[slot]).wait()
        off = (w + NW * i) * E_ECH
        pltpu.async_copy(hw.at[SV[slot]], G[b], SG[b])
        pltpu.async_copy(ew.at[pl.ds(off, E_ECH)], EV[b], SE[b])

    def consume(slot, b):
        pltpu.make_async_copy(ew.at[pl.ds(0, E_ECH)], G[b], SG[b]).wait()
        pltpu.make_async_copy(ew.at[pl.ds(0, E_ECH)], EV[b], SE[b]).wait()
        g_v = G[b]
        e_v = EV[b]

        def rowop(r, _):
            for jj in range(H // 16):
                sl = pl.ds(jj * 16, 16)
                g_v[r, sl] = jnp.maximum(g_v[r, sl] + e_v[r, sl], 0.0)
            return 0

        lax.fori_loop(0, E_ECH, rowop, 0)
        pltpu.sync_copy(g_v, acc_sh.at[DV[slot]], add=True)

    for i in range(4):
        prefetch(i, i)
    issue(0, 0, 0)
    issue(1, 1, 1)

    def step(j, _):
        for p in range(4):
            i = 4 * j + p
            b = p % 2
            consume(p, b)

            @pl.when(i + 4 < E_PER_TILE)
            def _():
                prefetch(i + 4, p)

            @pl.when(i + 2 < E_PER_TILE)
            def _():
                issue(i + 2, (p + 2) % 4, b)

        return 0

    lax.fori_loop(0, (E_PER_TILE - 1) // 4, step, 0)
    consume(0, 0)
    plsc.subcore_barrier()
    pltpu.sync_copy(acc_sh.at[pl.ds(row0, ROWS_PER_TILE)],
                    acc_out.at[c, pl.ds(row0, ROWS_PER_TILE)])


def _make_sc_edge():
    return pl.kernel(
        _sc_edge_body,
        out_type=jax.ShapeDtypeStruct((NC, N_PAD, H), F32),
        mesh=_mesh(),
        scratch_types=(
            pltpu.VMEM((E_ECH, H), F32),
            pltpu.VMEM((E_ECH, H), F32),
            pltpu.VMEM((E_ECH, H), F32),
            pltpu.VMEM((E_ECH, H), F32),
            pltpu.VMEM((E_ECH,), jnp.int32),
            pltpu.VMEM((E_ECH,), jnp.int32),
            pltpu.VMEM((E_ECH,), jnp.int32),
            pltpu.VMEM((E_ECH,), jnp.int32),
            pltpu.VMEM((E_ECH,), jnp.int32),
            pltpu.VMEM((E_ECH,), jnp.int32),
            pltpu.VMEM((E_ECH,), jnp.int32),
            pltpu.VMEM((E_ECH,), jnp.int32),
            pltpu.VMEM_SHARED((N_PAD, H), F32),
            pltpu.SemaphoreType.DMA,
            pltpu.SemaphoreType.DMA,
            pltpu.SemaphoreType.DMA,
            pltpu.SemaphoreType.DMA,
            pltpu.SemaphoreType.DMA,
            pltpu.SemaphoreType.DMA,
            pltpu.SemaphoreType.DMA,
            pltpu.SemaphoreType.DMA,
        ),
    )


def _sc_cnt_body(dst, cnt_out, d0, d1, ones_v, cnt_sh, si0, si1):
    c = lax.axis_index("c")
    s = lax.axis_index("s")
    w = c * NS + s
    row0 = s * ROWS_PER_TILE
    DV = (d0, d1)
    SI = (si0, si1)

    _zero_fill_rows(ones_v, CNT_ECH, H)
    for j in range(ROWS_PER_TILE // CNT_ECH):
        pltpu.sync_copy(ones_v, cnt_sh.at[pl.ds(row0 + j * CNT_ECH, CNT_ECH)])
    one = jnp.ones((16,), F32)

    def ofill(r, _):
        for jj in range(H // 16):
            ones_v[r, pl.ds(jj * 16, 16)] = one
        return 0

    lax.fori_loop(0, CNT_ECH, ofill, 0)
    plsc.subcore_barrier()

    def issue(i, b):
        ch = w + NW * i

        @pl.when(ch < CNT_NCH)
        def _():
            pltpu.async_copy(dst.at[pl.ds(ch * CNT_ECH, CNT_ECH)], DV[b], SI[b])

    def consume(i, b):
        ch = w + NW * i

        @pl.when(ch < CNT_NCH)
        def _():
            pltpu.make_async_copy(dst.at[pl.ds(0, CNT_ECH)], DV[b], SI[b]).wait()
            pltpu.sync_copy(ones_v, cnt_sh.at[DV[b]], add=True)

    issue(0, 0)
    issue(1, 1)

    def step(j, _):
        i = 2 * j
        consume(i, 0)
        issue(i + 2, 0)
        consume(i + 1, 1)
        issue(i + 3, 1)
        return 0

    lax.fori_loop(0, (CNT_ITERS + 1) // 2, step, 0)
    plsc.subcore_barrier()
    pltpu.sync_copy(cnt_sh.at[pl.ds(row0, ROWS_PER_TILE)],
                    cnt_out.at[c, pl.ds(row0, ROWS_PER_TILE)])


def _make_sc_cnt():
    return pl.kernel(
        _sc_cnt_body,
        out_type=jax.ShapeDtypeStruct((NC, N_PAD, H), F32),
        mesh=_mesh(),
        scratch_types=(
            pltpu.VMEM((CNT_ECH,), jnp.int32),
            pltpu.VMEM((CNT_ECH,), jnp.int32),
            pltpu.VMEM((CNT_ECH, H), F32),
            pltpu.VMEM_SHARED((N_PAD, H), F32),
            pltpu.SemaphoreType.DMA,
            pltpu.SemaphoreType.DMA,
        ),
    )


def _sc_od_body(hs, hd, sidx, didx, out,
                a0, b0, a1, b1, s0, d0, s1, d1, sa0, sb0, sa1, sb1):
    c = lax.axis_index("c")
    s = lax.axis_index("s")
    w = c * NS + s
    GA = (a0, a1)
    GB = (b0, b1)
    SV = (s0, s1)
    DV = (d0, d1)
    SA = (sa0, sa1)
    SB = (sb0, sb1)

    def issue(i, b):
        ch = w + NW * i

        @pl.when(ch < OD_CHUNKS)
        def _():
            off = ch * OD_ECH
            pltpu.sync_copy(sidx.at[pl.ds(off, OD_ECH)], SV[b])
            pltpu.sync_copy(didx.at[pl.ds(off, OD_ECH)], DV[b])
            pltpu.async_copy(hs.at[SV[b]], GA[b], SA[b])
            pltpu.async_copy(hd.at[DV[b]], GB[b], SB[b])

    def consume(i, b):
        ch = w + NW * i

        @pl.when(ch < OD_CHUNKS)
        def _():
            pltpu.make_async_copy(hs.at[pl.ds(0, OD_ECH)], GA[b], SA[b]).wait()
            pltpu.make_async_copy(hs.at[pl.ds(0, OD_ECH)], GB[b], SB[b]).wait()
            g1 = GA[b]
            g2 = GB[b]

            def rowop(r, _):
                for jj in range(H // 16):
                    sl = pl.ds(jj * 16, 16)
                    g1[r, sl] = g1[r, sl] + g2[r, sl]
                return 0

            lax.fori_loop(0, OD_ECH, rowop, 0)
            pltpu.sync_copy(g1, out.at[pl.ds(ch * OD_ECH, OD_ECH)])

    issue(0, 0)
    issue(1, 1)

    def step(j, _):
        i = 2 * j
        consume(i, 0)
        issue(i + 2, 0)
        consume(i + 1, 1)
        issue(i + 3, 1)
        return 0

    lax.fori_loop(0, (OD_ITERS + 1) // 2, step, 0)


def _make_sc_od():
    return pl.kernel(
        _sc_od_body,
        out_type=jax.ShapeDtypeStruct((OD_PAD, H), F32),
        mesh=_mesh(),
        scratch_types=(
            pltpu.VMEM((OD_ECH, H), F32),
            pltpu.VMEM((OD_ECH, H), F32),
            pltpu.VMEM((OD_ECH, H), F32),
            pltpu.VMEM((OD_ECH, H), F32),
            pltpu.VMEM((OD_ECH,), jnp.int32),
            pltpu.VMEM((OD_ECH,), jnp.int32),
            pltpu.VMEM((OD_ECH,), jnp.int32),
            pltpu.VMEM((OD_ECH,), jnp.int32),
            pltpu.SemaphoreType.DMA,
            pltpu.SemaphoreType.DMA,
            pltpu.SemaphoreType.DMA,
            pltpu.SemaphoreType.DMA,
        ),
    )


def _full(shape):
    return pl.BlockSpec(shape, lambda i: tuple(0 for _ in shape))


def _pad2(w, r, cdim):
    out = jnp.zeros((r, cdim), F32)
    return out.at[: w.shape[0], : w.shape[1]].set(w)


def kernel(node_features, edge_index, edge_features, od_features,
           od_src_idx, od_dst_idx, bottleneck_scores, flexdate_scores,
           params):
    p = params
    layers = p["layers"]
    src = edge_index[0]
    dst = edge_index[1]

    w1a = [l["msg_W1"][:H] for l in layers]
    w1b = [l["msg_W1"][H:] for l in layers]
    b1 = [l["msg_b1"].reshape(1, H) for l in layers]
    w2 = [l["msg_W2"] for l in layers]
    b2 = [l["msg_b2"].reshape(1, H) for l in layers]
    wua = [l["upd_W"][:H] for l in layers]
    wub = [l["upd_W"][H:] for l in layers]
    bu = [l["upd_b"].reshape(1, H) for l in layers]
    lng = [l["ln_g"].reshape(1, H) for l in layers]
    lnb = [l["ln_b"].reshape(1, H) for l in layers]

    wfull = _full((H, H))
    bfull = _full((1, H))
    nspec = pl.BlockSpec((NB, H), lambda i: (i, 0))

    h0, hw0 = pl.pallas_call(
        _a_node_body,
        grid=(N // NB,),
        in_specs=[nspec, wfull, bfull, wfull, bfull],
        out_specs=[nspec, nspec],
        out_shape=[jax.ShapeDtypeStruct((N, H), F32)] * 2,
    )(node_features, p["node_proj_W"], p["node_proj_b"].reshape(1, H),
      w1a[0], b1[0])

    espec = pl.BlockSpec((EB, 16), lambda i: (i, 0))
    eospec = pl.BlockSpec((EB, H), lambda i: (i, 0))

    def edge_proj(li):
        return pl.pallas_call(
            _a_edge_body,
            grid=(E // EB,),
            in_specs=[espec, _full((16, H // 2)), _full((1, H // 2)),
                      _full((H // 2, H))],
            out_specs=eospec,
            out_shape=jax.ShapeDtypeStruct((E, H), F32),
        )(edge_features, p["edge_proj_W"],
          p["edge_proj_b"].reshape(1, H // 2), w1b[li])

    sc_edge = _make_sc_edge()

    accspec = pl.BlockSpec((NC, NB, H), lambda i: (0, i, 0))
    cntspec = pl.BlockSpec((NC, NB, H), lambda i: (0, i, 0))

    def run_mid(h, acc, cnt, li):
        return pl.pallas_call(
            _b_mid_body,
            grid=(N // NB,),
            in_specs=[nspec, accspec, cntspec, wfull, bfull, wfull, wfull,
                      bfull, bfull, bfull, wfull, bfull],
            out_specs=[nspec, nspec],
            out_shape=[jax.ShapeDtypeStruct((N, H), F32)] * 2,
        )(h, acc, cnt, w2[li], b2[li], wua[li], wub[li], bu[li],
          lng[li], lnb[li], w1a[li + 1], b1[li + 1])

    cnt = _make_sc_cnt()(dst)
    acc0 = sc_edge(hw0, edge_proj(0), src, dst)
    h1, hw1 = run_mid(h0, acc0, cnt, 0)
    acc1 = sc_edge(hw1, edge_proj(1), src, dst)
    h2, hw2 = run_mid(h1, acc1, cnt, 1)
    acc2 = sc_edge(hw2, edge_proj(2), src, dst)

    odw1 = p["od_W1"]
    h3, hs, hd = pl.pallas_call(
        _b_last_body,
        grid=(N // NB,),
        in_specs=[nspec, accspec, cntspec, wfull, bfull, wfull, wfull,
                  bfull, bfull, bfull, wfull, wfull],
        out_specs=[nspec, nspec, nspec],
        out_shape=[jax.ShapeDtypeStruct((N, H), F32)] * 3,
    )(h2, acc2, cnt, w2[2], b2[2], wua[2], wub[2], bu[2], lng[2], lnb[2],
      odw1[:H], odw1[H:2 * H])

    pad = jnp.zeros((OD_PAD - NUM_OD,), jnp.int32)
    odsum = _make_sc_od()(hs, hd,
                   jnp.concatenate([od_src_idx, pad]),
                   jnp.concatenate([od_dst_idx, pad]))

    odf_p = jnp.pad(od_features, ((0, 0), (0, 6)))
    w1o_p = _pad2(odw1[2 * H:], 16, H)
    b3_p = jnp.broadcast_to(p["od_b3"].reshape(1, 1), (1, 8))
    cspec = pl.BlockSpec((CB, H), lambda i: (i, 0))
    corr8 = pl.pallas_call(
        _c_od_body,
        grid=(NUM_OD // CB,),
        in_specs=[cspec, pl.BlockSpec((CB, 16), lambda i: (i, 0)),
                  _full((16, H)), bfull, _full((H, H // 2)),
                  _full((1, H // 2)), _full((8, H // 2)), _full((1, 8))],
        out_specs=pl.BlockSpec((CB, 8), lambda i: (i, 0)),
        out_shape=jax.ShapeDtypeStruct((NUM_OD, 8), F32),
    )(odsum, odf_p, w1o_p, p["od_b1"].reshape(1, H), p["od_W2"],
      p["od_b2"].reshape(1, H // 2), _pad2(p["od_W3"].reshape(1, H // 2), 8, H // 2),
      b3_p)

    corr2 = corr8[:, 0].reshape(8, NUM_OD // 8)
    bn2 = bottleneck_scores.reshape(8, NUM_OD // 8)
    fl2 = flexdate_scores.reshape(8, NUM_OD // 8)
    vspec = _full((8, NUM_OD // 8))

    final2 = pl.pallas_call(
        _d_final_body,
        grid=(1,),
        in_specs=[_full((N, H)), vspec, vspec, vspec,
                  wfull, bfull, wfull, bfull, wfull, bfull, wfull, bfull,
                  bfull],
        out_specs=vspec,
        out_shape=jax.ShapeDtypeStruct((8, NUM_OD // 8), F32),
    )(h3, corr2, bn2, fl2,
      _pad2(p["blend_W1"], H, H), _pad2(p["blend_b1"].reshape(1, -1), 1, H),
      _pad2(p["blend_W2"], H, H), _pad2(p["blend_b2"].reshape(1, -1), 1, H),
      _pad2(p["conf_W1"], H, H), _pad2(p["conf_b1"].reshape(1, -1), 1, H),
      _pad2(p["conf_W2"], H, H), _pad2(p["conf_b2"].reshape(1, -1), 1, H),
      _pad2(p["log_alpha"].reshape(1, 1), 1, H))

    return final2.reshape(NUM_OD)


# speedup vs baseline: 1.3707x; 1.3707x over previous
import functools

import jax
import jax.numpy as jnp
from jax import lax
from jax.experimental import pallas as pl
from jax.experimental.pallas import tpu as pltpu
from jax.experimental.pallas import tpu_sc as plsc

N = 10000
E = 320000
H = 128
NUM_OD = 50000
F32 = jnp.float32

NC, NS = 2, 16
NW = NC * NS
N_PAD = 10240
ROWS_PER_TILE = N_PAD // NS

E_ECH = 80
E_NCH = E // E_ECH
E_PER_TILE = E_NCH // NW

CNT_ECH = 128
CNT_NCH = E // CNT_ECH
CNT_ITERS = -(-CNT_NCH // NW)

OD_ECH = 128
OD_PAD = 50048
OD_CHUNKS = OD_PAD // OD_ECH
OD_ITERS = -(-OD_CHUNKS // NW)

NB = 400
EB = 800
CB = 400

def _mesh():
    return plsc.VectorSubcoreMesh(
        core_axis_name="c", subcore_axis_name="s",
        num_cores=NC, num_subcores=NS)


def _dot(a, b):
    return jnp.dot(a, b, preferred_element_type=F32)


def _a_node_body(nf, wnp, bnp, w1a, b1, h_out, hw_out):
    h = jnp.maximum(_dot(nf[...], wnp[...]) + bnp[...], 0.0)
    h_out[...] = h
    hw_out[...] = _dot(h, w1a[...]) + b1[...]


def _a_edge_body(ef, wep, bep, wball, o0, o1, o2):
    e = jnp.maximum(_dot(ef[...], wep[...]) + bep[...], 0.0)
    eall = _dot(e, wball[...])
    o0[...] = eall[:, :H]
    o1[...] = eall[:, H:2 * H]
    o2[...] = eall[:, 2 * H:]


def _layer_update(h, acc, cnt, w2, b2, wua, wub, bu, g, beta):
    accsum = acc[0] + acc[1]
    c = cnt[0, :, 0:1] + cnt[1, :, 0:1]
    agg = (_dot(accsum, w2[...]) + c * b2[...]) / (c + 1e-8)
    upd = jnp.maximum(_dot(h, wua[...]) + _dot(agg, wub[...]) + bu[...], 0.0)
    x = upd + h
    mu = jnp.mean(x, axis=-1, keepdims=True)
    xc = x - mu
    var = jnp.mean(xc * xc, axis=-1, keepdims=True)
    return xc * lax.rsqrt(var + 1e-5) * g[...] + beta[...]


def _b_mid_body(h, acc, cnt, w2, b2, wua, wub, bu, g, beta, wn, bn,
                h_out, p_out):
    hn = _layer_update(h[...], acc[...], cnt[...], w2, b2, wua, wub, bu, g, beta)
    h_out[...] = hn
    p_out[...] = _dot(hn, wn[...]) + bn[...]


def _b_last_body(h, acc, cnt, w2, b2, wua, wub, bu, g, beta, ws, wd,
                 h_out, ps_out, pd_out):
    hn = _layer_update(h[...], acc[...], cnt[...], w2, b2, wua, wub, bu, g, beta)
    h_out[...] = hn
    ps_out[...] = _dot(hn, ws[...])
    pd_out[...] = _dot(hn, wd[...])


def _c_od_body(odsum, odf, w1o, b1, w2, b2, w3, b3, out):
    g1 = jnp.maximum(odsum[...] + _dot(odf[...], w1o[...]) + b1[...], 0.0)
    g2 = jnp.maximum(_dot(g1, w2[...]) + b2[...], 0.0)
    corr = jnp.sum(g2 * w3[...][0:1, :], axis=1, keepdims=True)
    out[...] = corr * jnp.ones((1, 8), F32) + b3[...]


def _d_final_body(h, corr2, bn2, fl2, bw1, bb1, bw2, bb2,
                  cw1, cb1, cw2, cb2, la, out):
    ge = jnp.sum(h[...], axis=0, keepdims=True) * (1.0 / N)
    t = jnp.maximum(_dot(ge, bw1[...]) + bb1[...], 0.0)
    logits = _dot(t, bw2[...]) + bb2[...]
    l0 = logits[0, 0]
    l1 = logits[0, 1]
    m = jnp.maximum(l0, l1)
    e0 = jnp.exp(l0 - m)
    e1 = jnp.exp(l1 - m)
    w0 = e0 / (e0 + e1)
    w1 = e1 / (e0 + e1)
    t2 = jnp.maximum(_dot(ge, cw1[...]) + cb1[...], 0.0)
    cv = _dot(t2, cw2[...]) + cb2[...]
    conf = 1.0 / (1.0 + jnp.exp(-cv[0, 0]))
    alpha = jnp.clip(jnp.exp(la[0, 0]), 0.0, 5.0)
    bnv = bn2[...]
    flv = fl2[...]
    crv = corr2[...]
    mb = jnp.max(jnp.abs(bnv))
    mf = jnp.max(jnp.abs(flv))
    mc = jnp.max(jnp.abs(crv))
    out[...] = (w0 * (bnv / (mb + 1e-12)) + w1 * (flv / (mf + 1e-12))
                + (conf * alpha) * (crv / (mc + 1e-12)))


def _fill_rows(ref, nrows, ncols, vec):
    def body(r, _):
        for jj in range(ncols // 16):
            ref[r, pl.ds(jj * 16, 16)] = vec
        return 0

    lax.fori_loop(0, nrows, body, 0)


def _sc_edge_body(with_count, hw, ew, src, dst, *refs):
    if with_count:
        (acc_out, cnt_out, g0, e0, g1, e1, s0, d0, s1, d1, s2, d2, s3, d3,
         acc_sh, sg0, se0, sg1, se1, si0, si1, si2, si3) = refs
    else:
        (acc_out, g0, e0, g1, e1, s0, d0, s1, d1, s2, d2, s3, d3,
         acc_sh, sg0, se0, sg1, se1, si0, si1, si2, si3) = refs
    c = lax.axis_index("c")
    s = lax.axis_index("s")
    w = c * NS + s
    row0 = s * ROWS_PER_TILE
    G = (g0, g1)
    EV = (e0, e1)
    SV = (s0, s1, s2, s3)
    DV = (d0, d1, d2, d3)
    SG = (sg0, sg1)
    SE = (se0, se1)
    SI = (si0, si1, si2, si3)

    if with_count:
        _fill_rows(g0, E_ECH, H, jnp.zeros((16,), F32))
        for j in range(ROWS_PER_TILE // E_ECH):
            pltpu.sync_copy(g0, acc_sh.at[pl.ds(row0 + j * E_ECH, E_ECH)])
        _fill_rows(g0, E_ECH, H, jnp.ones((16,), F32))
        plsc.subcore_barrier()

        def cprefetch(i, b):
            off = (w + NW * i) * E_ECH
            pltpu.async_copy(dst.at[pl.ds(off, E_ECH)], DV[b], SI[b])

        def cconsume(b):
            pltpu.make_async_copy(dst.at[pl.ds(0, E_ECH)], DV[b], SI[b]).wait()
            pltpu.sync_copy(g0, acc_sh.at[DV[b]], add=True)

        cprefetch(0, 0)
        cprefetch(1, 1)

        def cstep(j, _):
            cconsume(0)

            @pl.when(2 * j + 2 < E_PER_TILE)
            def _():
                cprefetch(2 * j + 2, 0)

            cconsume(1)

            @pl.when(2 * j + 3 < E_PER_TILE)
            def _():
                cprefetch(2 * j + 3, 1)

            return 0

        lax.fori_loop(0, (E_PER_TILE - 1) // 2, cstep, 0)
        cconsume(0)
        plsc.subcore_barrier()
        pltpu.sync_copy(acc_sh.at[pl.ds(row0, ROWS_PER_TILE)],
                        cnt_out.at[c, pl.ds(row0, ROWS_PER_TILE)])
        plsc.subcore_barrier()

    _fill_rows(g0, E_ECH, H, jnp.zeros((16,), F32))
    for j in range(ROWS_PER_TILE // E_ECH):
        pltpu.sync_copy(g0, acc_sh.at[pl.ds(row0 + j * E_ECH, E_ECH)])
    plsc.subcore_barrier()

    def prefetch(i, slot):
        off = (w + NW * i) * E_ECH
        pltpu.async_copy(src.at[pl.ds(off, E_ECH)], SV[slot], SI[slot])
        pltpu.async_copy(dst.at[pl.ds(off, E_ECH)], DV[slot], SI[slot])

    def issue(i, slot, b):
        pltpu.make_async_copy(src.at[pl.ds(0, E_ECH)], SV[slot], SI[slot]).wait()
        pltpu.make_async_copy(src.at[pl.ds(0, E_ECH)], DV[slot], SI[slot]).wait()
        off = (w + NW * i) * E_ECH
        pltpu.async_copy(hw.at[SV[slot]], G[b], SG[b])
        pltpu.async_copy(ew.at[pl.ds(off, E_ECH)], EV[b], SE[b])

    def consume(slot, b):
        pltpu.make_async_copy(ew.at[pl.ds(0, E_ECH)], G[b], SG[b]).wait()
        pltpu.make_async_copy(ew.at[pl.ds(0, E_ECH)], EV[b], SE[b]).wait()
        g_v = G[b]
        e_v = EV[b]

        def rowop(r, _):
            for jj in range(H // 16):
                sl = pl.ds(jj * 16, 16)
                g_v[r, sl] = jnp.maximum(g_v[r, sl] + e_v[r, sl], 0.0)
            return 0

        lax.fori_loop(0, E_ECH, rowop, 0)
        pltpu.sync_copy(g_v, acc_sh.at[DV[slot]], add=True)

    for i in range(4):
        prefetch(i, i)
    issue(0, 0, 0)
    issue(1, 1, 1)

    def step(j, _):
        for p in range(4):
            i = 4 * j + p
            b = p % 2
            consume(p, b)

            @pl.when(i + 4 < E_PER_TILE)
            def _():
                prefetch(i + 4, p)

            @pl.when(i + 2 < E_PER_TILE)
            def _():
                issue(i + 2, (p + 2) % 4, b)

        return 0

    lax.fori_loop(0, (E_PER_TILE - 1) // 4, step, 0)
    consume(0, 0)
    plsc.subcore_barrier()
    pltpu.sync_copy(acc_sh.at[pl.ds(row0, ROWS_PER_TILE)],
                    acc_out.at[c, pl.ds(row0, ROWS_PER_TILE)])


def _make_sc_edge(with_count=False):
    out_type = [jax.ShapeDtypeStruct((NC, N_PAD, H), F32)]
    if with_count:
        out_type.append(jax.ShapeDtypeStruct((NC, N_PAD, H), F32))
    return pl.kernel(
        functools.partial(_sc_edge_body, with_count),
        out_type=tuple(out_type) if with_count else out_type[0],
        mesh=_mesh(),
        scratch_types=(
            pltpu.VMEM((E_ECH, H), F32),
            pltpu.VMEM((E_ECH, H), F32),
            pltpu.VMEM((E_ECH, H), F32),
            pltpu.VMEM((E_ECH, H), F32),
            pltpu.VMEM((E_ECH,), jnp.int32),
            pltpu.VMEM((E_ECH,), jnp.int32),
            pltpu.VMEM((E_ECH,), jnp.int32),
            pltpu.VMEM((E_ECH,), jnp.int32),
            pltpu.VMEM((E_ECH,), jnp.int32),
            pltpu.VMEM((E_ECH,), jnp.int32),
            pltpu.VMEM((E_ECH,), jnp.int32),
            pltpu.VMEM((E_ECH,), jnp.int32),
            pltpu.VMEM_SHARED((N_PAD, H), F32),
            pltpu.SemaphoreType.DMA,
            pltpu.SemaphoreType.DMA,
            pltpu.SemaphoreType.DMA,
            pltpu.SemaphoreType.DMA,
            pltpu.SemaphoreType.DMA,
            pltpu.SemaphoreType.DMA,
            pltpu.SemaphoreType.DMA,
            pltpu.SemaphoreType.DMA,
        ),
    )


def _sc_od_body(hs, hd, sidx, didx, out,
                a0, b0, a1, b1, s0, d0, s1, d1, sa0, sb0, sa1, sb1):
    c = lax.axis_index("c")
    s = lax.axis_index("s")
    w = c * NS + s
    GA = (a0, a1)
    GB = (b0, b1)
    SV = (s0, s1)
    DV = (d0, d1)
    SA = (sa0, sa1)
    SB = (sb0, sb1)

    def issue(i, b):
        ch = w + NW * i

        @pl.when(ch < OD_CHUNKS)
        def _():
            off = ch * OD_ECH
            pltpu.sync_copy(sidx.at[pl.ds(off, OD_ECH)], SV[b])
            pltpu.sync_copy(didx.at[pl.ds(off, OD_ECH)], DV[b])
            pltpu.async_copy(hs.at[SV[b]], GA[b], SA[b])
            pltpu.async_copy(hd.at[DV[b]], GB[b], SB[b])

    def consume(i, b):
        ch = w + NW * i

        @pl.when(ch < OD_CHUNKS)
        def _():
            pltpu.make_async_copy(hs.at[pl.ds(0, OD_ECH)], GA[b], SA[b]).wait()
            pltpu.make_async_copy(hs.at[pl.ds(0, OD_ECH)], GB[b], SB[b]).wait()
            g1 = GA[b]
            g2 = GB[b]

            def rowop(r, _):
                for jj in range(H // 16):
                    sl = pl.ds(jj * 16, 16)
                    g1[r, sl] = g1[r, sl] + g2[r, sl]
                return 0

            lax.fori_loop(0, OD_ECH, rowop, 0)
            pltpu.sync_copy(g1, out.at[pl.ds(ch * OD_ECH, OD_ECH)])

    issue(0, 0)
    issue(1, 1)

    def step(j, _):
        i = 2 * j
        consume(i, 0)
        issue(i + 2, 0)
        consume(i + 1, 1)
        issue(i + 3, 1)
        return 0

    lax.fori_loop(0, (OD_ITERS + 1) // 2, step, 0)


def _make_sc_od():
    return pl.kernel(
        _sc_od_body,
        out_type=jax.ShapeDtypeStruct((OD_PAD, H), F32),
        mesh=_mesh(),
        scratch_types=(
            pltpu.VMEM((OD_ECH, H), F32),
            pltpu.VMEM((OD_ECH, H), F32),
            pltpu.VMEM((OD_ECH, H), F32),
            pltpu.VMEM((OD_ECH, H), F32),
            pltpu.VMEM((OD_ECH,), jnp.int32),
            pltpu.VMEM((OD_ECH,), jnp.int32),
            pltpu.VMEM((OD_ECH,), jnp.int32),
            pltpu.VMEM((OD_ECH,), jnp.int32),
            pltpu.SemaphoreType.DMA,
            pltpu.SemaphoreType.DMA,
            pltpu.SemaphoreType.DMA,
            pltpu.SemaphoreType.DMA,
        ),
    )


def _full(shape):
    return pl.BlockSpec(shape, lambda i: tuple(0 for _ in shape))


def _pad2(w, r, cdim):
    out = jnp.zeros((r, cdim), F32)
    return out.at[: w.shape[0], : w.shape[1]].set(w)


def kernel(node_features, edge_index, edge_features, od_features,
           od_src_idx, od_dst_idx, bottleneck_scores, flexdate_scores,
           params):
    p = params
    layers = p["layers"]
    src = edge_index[0]
    dst = edge_index[1]

    w1a = [l["msg_W1"][:H] for l in layers]
    w1b = [l["msg_W1"][H:] for l in layers]
    b1 = [l["msg_b1"].reshape(1, H) for l in layers]
    w2 = [l["msg_W2"] for l in layers]
    b2 = [l["msg_b2"].reshape(1, H) for l in layers]
    wua = [l["upd_W"][:H] for l in layers]
    wub = [l["upd_W"][H:] for l in layers]
    bu = [l["upd_b"].reshape(1, H) for l in layers]
    lng = [l["ln_g"].reshape(1, H) for l in layers]
    lnb = [l["ln_b"].reshape(1, H) for l in layers]

    wfull = _full((H, H))
    bfull = _full((1, H))
    nspec = pl.BlockSpec((NB, H), lambda i: (i, 0))

    h0, hw0 = pl.pallas_call(
        _a_node_body,
        grid=(N // NB,),
        in_specs=[nspec, wfull, bfull, wfull, bfull],
        out_specs=[nspec, nspec],
        out_shape=[jax.ShapeDtypeStruct((N, H), F32)] * 2,
    )(node_features, p["node_proj_W"], p["node_proj_b"].reshape(1, H),
      w1a[0], b1[0])

    espec = pl.BlockSpec((EB, 16), lambda i: (i, 0))
    eospec = pl.BlockSpec((EB, H), lambda i: (i, 0))

    ew = pl.pallas_call(
        _a_edge_body,
        grid=(E // EB,),
        in_specs=[espec, _full((16, H // 2)), _full((1, H // 2)),
                  _full((H // 2, 3 * H))],
        out_specs=[eospec] * 3,
        out_shape=[jax.ShapeDtypeStruct((E, H), F32)] * 3,
    )(edge_features, p["edge_proj_W"], p["edge_proj_b"].reshape(1, H // 2),
      jnp.concatenate([w1b[0], w1b[1], w1b[2]], axis=1))

    sc_edge = _make_sc_edge()

    accspec = pl.BlockSpec((NC, NB, H), lambda i: (0, i, 0))
    cntspec = pl.BlockSpec((NC, NB, H), lambda i: (0, i, 0))

    def run_mid(h, acc, cnt, li):
        return pl.pallas_call(
            _b_mid_body,
            grid=(N // NB,),
            in_specs=[nspec, accspec, cntspec, wfull, bfull, wfull, wfull,
                      bfull, bfull, bfull, wfull, bfull],
            out_specs=[nspec, nspec],
            out_shape=[jax.ShapeDtypeStruct((N, H), F32)] * 2,
        )(h, acc, cnt, w2[li], b2[li], wua[li], wub[li], bu[li],
          lng[li], lnb[li], w1a[li + 1], b1[li + 1])

    acc0, cnt = _make_sc_edge(True)(hw0, ew[0], src, dst)
    h1, hw1 = run_mid(h0, acc0, cnt, 0)
    acc1 = sc_edge(hw1, ew[1], src, dst)
    h2, hw2 = run_mid(h1, acc1, cnt, 1)
    acc2 = sc_edge(hw2, ew[2], src, dst)

    odw1 = p["od_W1"]
    h3, hs, hd = pl.pallas_call(
        _b_last_body,
        grid=(N // NB,),
        in_specs=[nspec, accspec, cntspec, wfull, bfull, wfull, wfull,
                  bfull, bfull, bfull, wfull, wfull],
        out_specs=[nspec, nspec, nspec],
        out_shape=[jax.ShapeDtypeStruct((N, H), F32)] * 3,
    )(h2, acc2, cnt, w2[2], b2[2], wua[2], wub[2], bu[2], lng[2], lnb[2],
      odw1[:H], odw1[H:2 * H])

    pad = jnp.zeros((OD_PAD - NUM_OD,), jnp.int32)
    odsum = _make_sc_od()(hs, hd,
                   jnp.concatenate([od_src_idx, pad]),
                   jnp.concatenate([od_dst_idx, pad]))

    odf_p = jnp.pad(od_features, ((0, 0), (0, 6)))
    w1o_p = _pad2(odw1[2 * H:], 16, H)
    b3_p = jnp.broadcast_to(p["od_b3"].reshape(1, 1), (1, 8))
    cspec = pl.BlockSpec((CB, H), lambda i: (i, 0))
    corr8 = pl.pallas_call(
        _c_od_body,
        grid=(NUM_OD // CB,),
        in_specs=[cspec, pl.BlockSpec((CB, 16), lambda i: (i, 0)),
                  _full((16, H)), bfull, _full((H, H // 2)),
                  _full((1, H // 2)), _full((8, H // 2)), _full((1, 8))],
        out_specs=pl.BlockSpec((CB, 8), lambda i: (i, 0)),
        out_shape=jax.ShapeDtypeStruct((NUM_OD, 8), F32),
    )(odsum, odf_p, w1o_p, p["od_b1"].reshape(1, H), p["od_W2"],
      p["od_b2"].reshape(1, H // 2), _pad2(p["od_W3"].reshape(1, H // 2), 8, H // 2),
      b3_p)

    corr2 = corr8[:, 0].reshape(8, NUM_OD // 8)
    bn2 = bottleneck_scores.reshape(8, NUM_OD // 8)
    fl2 = flexdate_scores.reshape(8, NUM_OD // 8)
    vspec = _full((8, NUM_OD // 8))

    final2 = pl.pallas_call(
        _d_final_body,
        grid=(1,),
        in_specs=[_full((N, H)), vspec, vspec, vspec,
                  wfull, bfull, wfull, bfull, wfull, bfull, wfull, bfull,
                  bfull],
        out_specs=vspec,
        out_shape=jax.ShapeDtypeStruct((8, NUM_OD // 8), F32),
    )(h3, corr2, bn2, fl2,
      _pad2(p["blend_W1"], H, H), _pad2(p["blend_b1"].reshape(1, -1), 1, H),
      _pad2(p["blend_W2"], H, H), _pad2(p["blend_b2"].reshape(1, -1), 1, H),
      _pad2(p["conf_W1"], H, H), _pad2(p["conf_b1"].reshape(1, -1), 1, H),
      _pad2(p["conf_W2"], H, H), _pad2(p["conf_b2"].reshape(1, -1), 1, H),
      _pad2(p["log_alpha"].reshape(1, 1), 1, H))

    return final2.reshape(NUM_OD)
